# baseline (speedup 1.0000x reference)
import functools

import jax
import jax.numpy as jnp
from jax.experimental import pallas as pl
from jax.experimental.pallas import tpu as pltpu

Z_SPLIT = 32
D_TOT = 64
M_CODES = 1024
TOKEN_BLOCK = 4096


def _fused_kernel(x_ref, m_ref, w1_ref, g_ref, b_ref, w2_ref, b2_ref,
                  emb_ref, z_ref, q_ref, loss_ref, *, inv_b):
    tb = x_ref.shape[0]
    ch = w1_ref.shape[1]
    h = jnp.dot(x_ref[...], w1_ref[...], preferred_element_type=jnp.float32)
    ones_c = jnp.full((ch, 1), 1.0 / ch, dtype=jnp.float32)
    mu = jnp.dot(h, ones_c, preferred_element_type=jnp.float32)
    ms = jnp.dot(h * h, ones_c, preferred_element_type=jnp.float32)
    var = ms - mu * mu
    h = (h - mu) * jax.lax.rsqrt(var + 1e-5) * g_ref[...] + b_ref[...]
    h = jnp.maximum(h, 0.0)
    z = jnp.dot(h, w2_ref[...], preferred_element_type=jnp.float32) + b2_ref[...]
    z_ref[...] = z

    lane = jax.lax.broadcasted_iota(jnp.int32, (tb, D_TOT), 1)
    head0 = lane < Z_SPLIT
    neg_inf = jnp.float32(-jnp.inf)
    m0 = jnp.max(jnp.where(head0, z, neg_inf), axis=-1, keepdims=True)
    m1 = jnp.max(jnp.where(head0, neg_inf, z), axis=-1, keepdims=True)
    mx = jnp.where(head0, m0, m1)
    ez = jnp.exp(z - mx)
    s0 = jnp.sum(jnp.where(head0, ez, 0.0), axis=-1, keepdims=True)
    s1 = jnp.sum(jnp.where(head0, 0.0, ez), axis=-1, keepdims=True)
    et = ez * jnp.where(head0, 1.0 / s0, 1.0 / s1)
    const = (jnp.sum(et * z, axis=-1, keepdims=True)
             - m0 - jnp.log(s0) - m1 - jnp.log(s1))

    log_e = jnp.log(emb_ref[:, :D_TOT])
    dots = jax.lax.dot_general(et, log_e, (((1,), (1,)), ((), ())),
                               preferred_element_type=jnp.float32)
    maxdots = jnp.max(dots, axis=-1, keepdims=True)
    minval = const - maxdots
    eq = (dots >= maxdots).astype(jnp.float32)
    q_aug = jnp.dot(eq, emb_ref[...],
                    preferred_element_type=jnp.float32)
    cnt = q_aug[:, D_TOT:D_TOT + 1]
    q_ref[...] = q_aug[:, :D_TOT] / cnt

    contrib = jnp.sum(minval * m_ref[...], axis=(0, 1),
                      keepdims=True) * (0.25 * inv_b)

    @pl.when(pl.program_id(0) == 0)
    def _zero():
        loss_ref[...] = jnp.zeros_like(loss_ref)

    loss_ref[...] += contrib


def kernel(x, masks, W1, ln_g, ln_b, W2, b2, embedding):
    B, T, Cin = x.shape
    Ch = W1.shape[0]
    M, D = embedding.shape
    N = B * T
    xf = x.reshape(N, Cin)
    mf = masks.reshape(N, 1)
    nblk = N // TOKEN_BLOCK
    grid = (nblk,)

    pc = pl.pallas_call(
        functools.partial(_fused_kernel, inv_b=1.0 / B),
        grid=grid,
        in_specs=[
            pl.BlockSpec((TOKEN_BLOCK, Cin), lambda i: (i, 0)),
            pl.BlockSpec((TOKEN_BLOCK, 1), lambda i: (i, 0)),
            pl.BlockSpec((Cin, Ch), lambda i: (0, 0)),
            pl.BlockSpec((1, Ch), lambda i: (0, 0)),
            pl.BlockSpec((1, Ch), lambda i: (0, 0)),
            pl.BlockSpec((Ch, D), lambda i: (0, 0)),
            pl.BlockSpec((1, D), lambda i: (0, 0)),
            pl.BlockSpec((M, D + 1), lambda i: (0, 0)),
        ],
        out_specs=[
            pl.BlockSpec((TOKEN_BLOCK, D), lambda i: (i, 0)),
            pl.BlockSpec((TOKEN_BLOCK, D), lambda i: (i, 0)),
            pl.BlockSpec((1, 1), lambda i: (0, 0)),
        ],
        out_shape=[
            jax.ShapeDtypeStruct((N, D), jnp.float32),
            jax.ShapeDtypeStruct((N, D), jnp.float32),
            jax.ShapeDtypeStruct((1, 1), jnp.float32),
        ],
        compiler_params=pltpu.CompilerParams(
            dimension_semantics=("arbitrary",)),
    )
    emb_aug = jnp.concatenate(
        [embedding, jnp.ones((M, 1), jnp.float32)], axis=1)
    out = pc(xf, mf, W1.T, ln_g.reshape(1, Ch), ln_b.reshape(1, Ch),
             W2.T, b2.reshape(1, D), emb_aug)
    z_flat, q_flat, loss_parts = out

    z = z_flat.reshape(B, T, D)
    q = q_flat.reshape(B, T, D)
    return (z, q, loss_parts.reshape(()))


# speedup vs baseline: 1.2027x; 1.2027x over previous
import functools

import jax
import jax.numpy as jnp
from jax.experimental import pallas as pl
from jax.experimental.pallas import tpu as pltpu

Z_SPLIT = 32
D_TOT = 64
M_CODES = 1024
TOKEN_BLOCK = 4096


def _fused_kernel(x_ref, m_ref, w1_ref, g_ref, b_ref, w2_ref, b2_ref,
                  emb_ref, z_ref, q_ref, loss_ref, *, inv_b):
    tb = x_ref.shape[0]
    h = jnp.dot(x_ref[...], w1_ref[...], preferred_element_type=jnp.float32)
    mu = jnp.mean(h, axis=-1, keepdims=True)
    var = jnp.mean((h - mu) ** 2, axis=-1, keepdims=True)
    h = (h - mu) * jax.lax.rsqrt(var + 1e-5) * g_ref[...] + b_ref[...]
    h = jnp.maximum(h, 0.0)
    z = jnp.dot(h, w2_ref[...], preferred_element_type=jnp.float32) + b2_ref[...]
    z_ref[...] = z

    lane = jax.lax.broadcasted_iota(jnp.int32, (tb, D_TOT), 1)
    head0 = lane < Z_SPLIT
    neg_inf = jnp.float32(-jnp.inf)
    m0 = jnp.max(jnp.where(head0, z, neg_inf), axis=-1, keepdims=True)
    m1 = jnp.max(jnp.where(head0, neg_inf, z), axis=-1, keepdims=True)
    mx = jnp.where(head0, m0, m1)
    ez = jnp.exp(z - mx)
    s0 = jnp.sum(jnp.where(head0, ez, 0.0), axis=-1, keepdims=True)
    s1 = jnp.sum(jnp.where(head0, 0.0, ez), axis=-1, keepdims=True)
    et = ez * jnp.where(head0, 1.0 / s0, 1.0 / s1)
    const = (jnp.sum(et * z, axis=-1, keepdims=True)
             - m0 - jnp.log(s0) - m1 - jnp.log(s1))

    log_e = jnp.log(emb_ref[:, :D_TOT])
    dots = jax.lax.dot_general(et, log_e, (((1,), (1,)), ((), ())),
                               preferred_element_type=jnp.float32)
    maxdots = jnp.max(dots, axis=-1, keepdims=True)
    minval = const - maxdots
    eq = (dots >= maxdots).astype(jnp.float32)
    q_aug = jnp.dot(eq, emb_ref[...],
                    preferred_element_type=jnp.float32)
    cnt = q_aug[:, D_TOT:D_TOT + 1]
    q_ref[...] = q_aug[:, :D_TOT] / cnt

    contrib = jnp.sum(minval * m_ref[...], axis=(0, 1),
                      keepdims=True) * (0.25 * inv_b)

    @pl.when(pl.program_id(0) == 0)
    def _zero():
        loss_ref[...] = jnp.zeros_like(loss_ref)

    loss_ref[...] += contrib


def kernel(x, masks, W1, ln_g, ln_b, W2, b2, embedding):
    B, T, Cin = x.shape
    Ch = W1.shape[0]
    M, D = embedding.shape
    N = B * T
    xf = x.reshape(N, Cin)
    mf = masks.reshape(N, 1)
    nblk = N // TOKEN_BLOCK
    grid = (nblk,)

    pc = pl.pallas_call(
        functools.partial(_fused_kernel, inv_b=1.0 / B),
        grid=grid,
        in_specs=[
            pl.BlockSpec((TOKEN_BLOCK, Cin), lambda i: (i, 0)),
            pl.BlockSpec((TOKEN_BLOCK, 1), lambda i: (i, 0)),
            pl.BlockSpec((Cin, Ch), lambda i: (0, 0)),
            pl.BlockSpec((1, Ch), lambda i: (0, 0)),
            pl.BlockSpec((1, Ch), lambda i: (0, 0)),
            pl.BlockSpec((Ch, D), lambda i: (0, 0)),
            pl.BlockSpec((1, D), lambda i: (0, 0)),
            pl.BlockSpec((M, D + 1), lambda i: (0, 0)),
        ],
        out_specs=[
            pl.BlockSpec((TOKEN_BLOCK, D), lambda i: (i, 0)),
            pl.BlockSpec((TOKEN_BLOCK, D), lambda i: (i, 0)),
            pl.BlockSpec((1, 1), lambda i: (0, 0)),
        ],
        out_shape=[
            jax.ShapeDtypeStruct((N, D), jnp.float32),
            jax.ShapeDtypeStruct((N, D), jnp.float32),
            jax.ShapeDtypeStruct((1, 1), jnp.float32),
        ],
        compiler_params=pltpu.CompilerParams(
            dimension_semantics=("arbitrary",)),
    )
    emb_aug = jnp.concatenate(
        [embedding, jnp.ones((M, 1), jnp.float32)], axis=1)
    out = pc(xf, mf, W1.T, ln_g.reshape(1, Ch), ln_b.reshape(1, Ch),
             W2.T, b2.reshape(1, D), emb_aug)
    z_flat, q_flat, loss_parts = out

    z = z_flat.reshape(B, T, D)
    q = q_flat.reshape(B, T, D)
    return (z, q, loss_parts.reshape(()))
